# baseline (speedup 1.0000x reference)
import numpy as np

import jax
import jax.numpy as jnp
from jax.experimental import pallas as pl

_N = 8192
_D = 768
_MT = 16
_MY = 32
_BLK = 2048
_C = _MT * _MY

_FOLD_NP = np.zeros((_C, 2 * _MY), np.float32)
for _l in range(_C):
    _FOLD_NP[_l, _l % _MY] = 1.0
_FOLD_NP[:, _MY:] = 1.0


def _body(x_ref, t_ref, w_ref, b_ref, f_ref, o_ref):
    tt = t_ref[...].T
    lane_task = jax.lax.broadcasted_iota(jnp.int32, (1, _C), 1) // _MY
    w = w_ref[...].reshape(_C, _D)
    brow = jnp.concatenate(
        [b_ref[e:e + 1, :] for e in range(_MT)], axis=1
    )
    logits = jax.lax.dot_general(
        x_ref[...], w, (((1,), (1,)), ((), ())),
        preferred_element_type=jnp.float32,
    ) + brow
    pe = jnp.where(lane_task == tt, jnp.exp(logits), 0.0)
    y = jax.lax.dot_general(
        pe, f_ref[...], (((1,), (0,)), ((), ())), preferred_element_type=jnp.float32
    )
    yt = y.T
    o_ref[...] = yt[:_MY, :] / yt[_MY:, :]


def kernel(x, t, W, b):
    n, d = x.shape
    t2 = t.reshape(1, n)
    fold = jnp.asarray(_FOLD_NP)
    grid = (n // _BLK,)
    out = pl.pallas_call(
        _body,
        grid=grid,
        in_specs=[
            pl.BlockSpec((_BLK, d), lambda i: (i, 0)),
            pl.BlockSpec((1, _BLK), lambda i: (0, i)),
            pl.BlockSpec((_MT, _MY, d), lambda i: (0, 0, 0)),
            pl.BlockSpec((_MT, _MY), lambda i: (0, 0)),
            pl.BlockSpec((_C, 2 * _MY), lambda i: (0, 0)),
        ],
        out_specs=pl.BlockSpec((_MY, _BLK), lambda i: (0, i)),
        out_shape=jax.ShapeDtypeStruct((_MY, n), x.dtype),
    )(x, t2, W, b, fold)
    return out.T
